# baseline (speedup 1.0000x reference)
import functools

import jax
import jax.numpy as jnp
from jax import lax
from jax.experimental import pallas as pl
from jax.experimental.pallas import tpu as pltpu
from jax.experimental.pallas import tpu_sc as plsc

C_IN = 100000
D_MODEL = 64
D_PAD = 128
BATCH = 4096
SEQ = 200
LANES = 16

_info = plsc.get_sparse_core_info()
NC = _info.num_cores
NS = _info.num_subcores
NW = NC * NS
ROWS_PER_W = BATCH // NW
NBUF = 2


def _gather_kernel(x_hbm, w_hbm, out_hbm, idx_v0, idx_v1, rows_v, pack_v,
                   sem_idx, sem_g, sem_w):
    idx_v = (idx_v0, idx_v1)
    wid = lax.axis_index("s") * NC + lax.axis_index("c")
    base = wid * ROWS_PER_W

    def start_idx(g, b):
        pltpu.async_copy(x_hbm.at[base + g], idx_v[b], sem_idx.at[b])

    def start_gather(g, b):
        pltpu.make_async_copy(
            x_hbm.at[base + g], idx_v[b], sem_idx.at[b]).wait()
        pltpu.async_copy(w_hbm.at[idx_v[b]], rows_v.at[b], sem_g.at[b])

    def repack(b):
        def row_body(i, carry):
            for k in range(D_MODEL // LANES):
                pack_v[b, i, pl.ds(k * LANES, LANES)] = (
                    rows_v[b, i, pl.ds(k * LANES, LANES)])
            return carry
        lax.fori_loop(0, SEQ, row_body, 0)

    start_idx(0, 0)
    start_idx(1, 1)
    start_gather(0, 0)

    def body(s, carry):
        for b in range(NBUF):
            g = s * NBUF + b
            nb = (b + 1) % NBUF
            pltpu.make_async_copy(
                w_hbm.at[idx_v[b]], rows_v.at[b], sem_g.at[b]).wait()
            @pl.when(g + 1 < ROWS_PER_W)
            def _():
                start_gather(g + 1, nb)
            @pl.when(g + 2 < ROWS_PER_W)
            def _():
                start_idx(g + 2, b)
            @pl.when(g >= NBUF)
            def _():
                pltpu.make_async_copy(
                    pack_v.at[b], out_hbm.at[base + g - NBUF],
                    sem_w.at[b]).wait()
            repack(b)
            pltpu.async_copy(pack_v.at[b], out_hbm.at[base + g], sem_w.at[b])
        return carry

    lax.fori_loop(0, ROWS_PER_W // NBUF, body, 0)

    for t in range(NBUF):
        g = ROWS_PER_W - NBUF + t
        pltpu.make_async_copy(
            pack_v.at[g % NBUF], out_hbm.at[base + g],
            sem_w.at[g % NBUF]).wait()


@jax.jit
def _embed(x, W):
    w_pad = jnp.pad(W, ((0, 0), (0, D_PAD - D_MODEL)))
    mesh = plsc.VectorSubcoreMesh(core_axis_name="c", subcore_axis_name="s")
    gather = functools.partial(
        pl.kernel,
        mesh=mesh,
        out_type=jax.ShapeDtypeStruct((BATCH, SEQ, D_MODEL), jnp.float32),
        scratch_types=[
            pltpu.VMEM((SEQ,), jnp.int32),
            pltpu.VMEM((SEQ,), jnp.int32),
            pltpu.VMEM((NBUF, SEQ, D_PAD), jnp.float32),
            pltpu.VMEM((NBUF, SEQ, D_MODEL), jnp.float32),
            pltpu.SemaphoreType.DMA((NBUF,)),
            pltpu.SemaphoreType.DMA((NBUF,)),
            pltpu.SemaphoreType.DMA((NBUF,)),
        ],
    )(_gather_kernel)
    return gather(x, w_pad)


def kernel(x, W):
    return _embed(x, W)


# speedup vs baseline: 1.4637x; 1.4637x over previous
import functools

import jax
import jax.numpy as jnp
from jax import lax
from jax.experimental import pallas as pl
from jax.experimental.pallas import tpu as pltpu
from jax.experimental.pallas import tpu_sc as plsc

C_IN = 100000
D_MODEL = 64
W32 = D_MODEL
BATCH = 4096
SEQ = 200
ROW_WORDS = SEQ * W32
STAGE_MINOR = 128
STAGE_ROWS_PER_CHUNK = ROW_WORDS // STAGE_MINOR

_info = plsc.get_sparse_core_info()
NC = _info.num_cores
NS = _info.num_subcores
NW = NC * NS
ROWS_PER_W = BATCH // NW
NBUF = 2


def _gather_kernel(x_hbm, w_hbm, stage_hbm, idx_v0, idx_v1, rows_v,
                   sem_idx, sem_g, sem_w):
    idx_v = (idx_v0, idx_v1)
    wid = lax.axis_index("s") * NC + lax.axis_index("c")
    base = wid * ROWS_PER_W

    def stage_slot(r):
        return stage_hbm.at[r, :, pl.ds(0, D_MODEL)]

    for b in range(NBUF):
        pltpu.async_copy(x_hbm.at[base + b], idx_v[b], sem_idx.at[b])

    def super_body(s, carry):
        for b in range(NBUF):
            g = s * NBUF + b
            r = base + g
            src = rows_v.at[b]
            @pl.when(s > 0)
            def _():
                pltpu.make_async_copy(
                    src, stage_slot(r - NBUF), sem_w.at[b]).wait()
            pltpu.make_async_copy(
                x_hbm.at[r], idx_v[b], sem_idx.at[b]).wait()
            pltpu.async_copy(w_hbm.at[idx_v[b]], rows_v.at[b],
                             sem_g.at[b]).wait()
            @pl.when(g + NBUF < ROWS_PER_W)
            def _():
                pltpu.async_copy(
                    x_hbm.at[r + NBUF], idx_v[b], sem_idx.at[b])
            pltpu.async_copy(src, stage_slot(r), sem_w.at[b])
        return carry

    lax.fori_loop(0, ROWS_PER_W // NBUF, super_body, 0)

    for b in range(NBUF):
        r = base + ROWS_PER_W - NBUF + b
        pltpu.make_async_copy(rows_v.at[b], stage_slot(r), sem_w.at[b]).wait()


@jax.jit
def _embed(x, W):
    mesh = plsc.VectorSubcoreMesh(core_axis_name="c", subcore_axis_name="s")
    gather = functools.partial(
        pl.kernel,
        mesh=mesh,
        out_type=jax.ShapeDtypeStruct(
            (BATCH, SEQ, STAGE_MINOR), jnp.float32),
        scratch_types=[
            pltpu.VMEM((SEQ,), jnp.int32),
            pltpu.VMEM((SEQ,), jnp.int32),
            pltpu.VMEM((NBUF, SEQ, D_MODEL), jnp.float32),
            pltpu.SemaphoreType.DMA((NBUF,)),
            pltpu.SemaphoreType.DMA((NBUF,)),
            pltpu.SemaphoreType.DMA((NBUF,)),
        ],
        compiler_params=pltpu.CompilerParams(use_tc_tiling_on_sc=False),
    )(_gather_kernel)
    stage = gather(x, W)
    return stage[:, :, :D_MODEL]


def kernel(x, W):
    return _embed(x, W)


# speedup vs baseline: 1.6274x; 1.1118x over previous
import functools

import jax
import jax.numpy as jnp
from jax import lax
from jax.experimental import pallas as pl
from jax.experimental.pallas import tpu as pltpu
from jax.experimental.pallas import tpu_sc as plsc

C_IN = 100000
D_MODEL = 64
STAGE_MINOR = 128
BATCH = 4096
SEQ = 200
XROWS_PER_CHUNK = 4
CHUNK = XROWS_PER_CHUNK * SEQ

_info = plsc.get_sparse_core_info()
NC = _info.num_cores
NS = _info.num_subcores
NW = NC * NS
CH_PER_W = BATCH // (XROWS_PER_CHUNK * NW)
NBUF = 2


def _gather_kernel(x_hbm, w_hbm, stage_hbm, idx_v0, idx_v1, rows_v,
                   sem_idx, sem_g, sem_w):
    idx_v = (idx_v0, idx_v1)
    wid = lax.axis_index("s") * NC + lax.axis_index("c")
    base = wid * CH_PER_W

    def stage_slot(c):
        return stage_hbm.at[pl.ds(CHUNK * c, CHUNK), pl.ds(0, D_MODEL)]

    def start_idx(c, b):
        for j in range(XROWS_PER_CHUNK):
            pltpu.async_copy(
                x_hbm.at[XROWS_PER_CHUNK * c + j],
                idx_v[b].at[pl.ds(SEQ * j, SEQ)], sem_idx.at[b])

    def wait_idx(c, b):
        for j in range(XROWS_PER_CHUNK):
            pltpu.make_async_copy(
                x_hbm.at[XROWS_PER_CHUNK * c + j],
                idx_v[b].at[pl.ds(SEQ * j, SEQ)], sem_idx.at[b]).wait()

    for b in range(NBUF):
        start_idx(base + b, b)

    def super_body(s, carry):
        for b in range(NBUF):
            g = s * NBUF + b
            c = base + g
            src = rows_v.at[b]
            @pl.when(s > 0)
            def _():
                pltpu.make_async_copy(
                    src, stage_slot(c - NBUF), sem_w.at[b]).wait()
            wait_idx(c, b)
            pltpu.async_copy(w_hbm.at[idx_v[b]], rows_v.at[b],
                             sem_g.at[b]).wait()
            @pl.when(g + NBUF < CH_PER_W)
            def _():
                start_idx(c + NBUF, b)
            pltpu.async_copy(src, stage_slot(c), sem_w.at[b])
        return carry

    lax.fori_loop(0, CH_PER_W // NBUF, super_body, 0)

    for b in range(NBUF):
        c = base + CH_PER_W - NBUF + b
        pltpu.make_async_copy(rows_v.at[b], stage_slot(c), sem_w.at[b]).wait()


@jax.jit
def _embed(x, W):
    mesh = plsc.VectorSubcoreMesh(core_axis_name="c", subcore_axis_name="s")
    gather = functools.partial(
        pl.kernel,
        mesh=mesh,
        out_type=jax.ShapeDtypeStruct(
            (BATCH * SEQ, STAGE_MINOR), jnp.float32),
        scratch_types=[
            pltpu.VMEM((CHUNK,), jnp.int32),
            pltpu.VMEM((CHUNK,), jnp.int32),
            pltpu.VMEM((NBUF, CHUNK, D_MODEL), jnp.float32),
            pltpu.SemaphoreType.DMA((NBUF,)),
            pltpu.SemaphoreType.DMA((NBUF,)),
            pltpu.SemaphoreType.DMA((NBUF,)),
        ],
        compiler_params=pltpu.CompilerParams(use_tc_tiling_on_sc=False),
    )(_gather_kernel)
    stage = gather(x, W)
    return stage[:, :D_MODEL].reshape(BATCH, SEQ, D_MODEL)


def kernel(x, W):
    return _embed(x, W)
